# baseline (speedup 1.0000x reference)
import functools

import jax
import jax.numpy as jnp
from jax import lax
from jax.experimental import pallas as pl
from jax.experimental.pallas import tpu as pltpu
from jax.experimental.pallas import tpu_sc as plsc

_N = 16384
_S = 1024
_K = 16
_C = 256
_EPS = 1e-5
_G = 128
_QB = 64
_RB = 512
_INF = float("inf")
_BIGI = 2**30


def _fps_body(pts_ref, idx_ref, coord_ref):
    px = pts_ref[0]
    py = pts_ref[1]
    pz = pts_ref[2]
    flat = (lax.broadcasted_iota(jnp.int32, (_G, _G), 0) * _G
            + lax.broadcasted_iota(jnp.int32, (_G, _G), 1))
    sub8 = lax.broadcasted_iota(jnp.int32, (8, _G), 0)
    lane8 = lax.broadcasted_iota(jnp.int32, (8, _G), 1)

    def red2(x, red):
        return red(red(x, axis=0, keepdims=True), axis=1, keepdims=True)

    lx = px[0:1, 0:1]
    ly = py[0:1, 0:1]
    lz = pz[0:1, 0:1]
    idx_ref[...] = jnp.zeros((8, _G), jnp.int32)
    c0 = jnp.where(sub8 + lane8 == 0, lx, 0.0)
    c1 = jnp.where(sub8 + lane8 == 0, ly, 0.0)
    c2 = jnp.where(sub8 + lane8 == 0, lz, 0.0)

    def body(i, carry):
        dists, lx, ly, lz = carry
        dx = px - lx
        dy = py - ly
        dz = pz - lz
        d = dx * dx + dy * dy + dz * dz
        dists = jnp.minimum(dists, d)
        m = red2(dists, jnp.max)
        idxv = red2(jnp.where(dists == m, flat, _BIGI), jnp.min)
        sel = flat == idxv
        nx = red2(jnp.where(sel, px, -_INF), jnp.max)
        ny = red2(jnp.where(sel, py, -_INF), jnp.max)
        nz = red2(jnp.where(sel, pz, -_INF), jnp.max)
        sel2 = (sub8 == i // _G) & (lane8 == i % _G)
        idx_ref[...] = jnp.where(sel2, idxv, idx_ref[...])
        coord_ref[0] = jnp.where(sel2, nx, coord_ref[0])
        coord_ref[1] = jnp.where(sel2, ny, coord_ref[1])
        coord_ref[2] = jnp.where(sel2, nz, coord_ref[2])
        return dists, nx, ny, nz

    coord_ref[0] = c0
    coord_ref[1] = c1
    coord_ref[2] = c2
    dists0 = jnp.full((_G, _G), _INF)
    lax.fori_loop(1, _S, body, (dists0, lx, ly, lz))


def _run_fps(pts3):
    return pl.pallas_call(
        _fps_body,
        out_shape=[
            jax.ShapeDtypeStruct((8, _G), jnp.int32),
            jax.ShapeDtypeStruct((3, 8, _G), jnp.float32),
        ],
    )(pts3)


def _topk_body(sq_ref, qb_ref, pts_ref, psq_ref, out_ref, s_ref):
    sq = sq_ref[:, 0:1]
    qb = qb_ref[...]

    def cbody(c, _):
        m = jnp.dot(qb, pts_ref[c], preferred_element_type=jnp.float32)
        s_ref[c] = (sq - 2.0 * m) + psq_ref[c]
        return 0

    lax.fori_loop(0, _N // _G, cbody, 0)

    _U, _A = 8, 2
    lane = lax.broadcasted_iota(jnp.int32, (_QB, _G), 1)
    lv = jnp.full((_QB, 1), -_INF)
    for j in range(_K):
        def fbody(t, st, lv=lv):
            bvs, bcs = list(st[0]), list(st[1])
            for u in range(_U):
                c = t * _U + u
                v = s_ref[c]
                vv = jnp.where(v > lv, v, _INF)
                a = u % _A
                take = vv < bvs[a]
                bvs[a] = jnp.minimum(bvs[a], vv)
                bcs[a] = jnp.where(take, c, bcs[a])
            return tuple(bvs), tuple(bcs)

        bvs, bcs = lax.fori_loop(
            0, _N // _G // _U, fbody,
            (tuple(jnp.full((_QB, _G), _INF) for _ in range(_A)),
             tuple(jnp.zeros((_QB, _G), jnp.int32) for _ in range(_A))))
        bv, bc = bvs[0], bcs[0]
        for a in range(1, _A):
            take = (bvs[a] < bv) | ((bvs[a] == bv) & (bcs[a] < bc))
            bv = jnp.minimum(bv, bvs[a])
            bc = jnp.where(take, bcs[a], bc)
        rv = jnp.min(bv, axis=1, keepdims=True)
        gi = bc * _G + lane
        ri = jnp.min(jnp.where(bv == rv, gi, _BIGI), axis=1, keepdims=True)
        out_ref[:, j:j + 1] = ri
        lv = rv


def _run_topk(sq, qb, ptsb3, psq3):
    return pl.pallas_call(
        _topk_body,
        grid=(_S // _QB,),
        in_specs=[
            pl.BlockSpec((_QB, 1), lambda i: (i, 0)),
            pl.BlockSpec((_QB, 8), lambda i: (i, 0)),
            pl.BlockSpec((_G, 8, _G), lambda i: (0, 0, 0)),
            pl.BlockSpec((_G, 1, _G), lambda i: (0, 0, 0)),
        ],
        out_specs=pl.BlockSpec((_QB, _K), lambda i: (i, 0)),
        out_shape=jax.ShapeDtypeStruct((_S, _K), jnp.int32),
        scratch_shapes=[pltpu.VMEM((_G, _QB, _G), jnp.float32)],
    )(sq, qb, ptsb3, psq3)


def _make_gather():
    info = plsc.get_sparse_core_info()
    nw = info.num_cores * info.num_subcores
    b_per_w = (_S * _K) // nw
    chunk = 128
    mesh = plsc.VectorSubcoreMesh(core_axis_name="c", subcore_axis_name="s")

    @functools.partial(
        pl.kernel, mesh=mesh,
        out_type=jax.ShapeDtypeStruct((_S * _K, _C), jnp.float32),
        scratch_types=[
            pltpu.VMEM((chunk,), jnp.int32),
            pltpu.VMEM((chunk, _C), jnp.float32),
            pltpu.SemaphoreType.DMA,
        ],
    )
    def gather(table_hbm, idx_hbm, out_hbm, idx_v, rows_v, sem):
        wid = lax.axis_index("s") * info.num_cores + lax.axis_index("c")
        base = wid * b_per_w
        for h in range(b_per_w // chunk):
            off = base + h * chunk
            pltpu.sync_copy(idx_hbm.at[pl.ds(off, chunk)], idx_v)
            pltpu.async_copy(table_hbm.at[idx_v], rows_v, sem).wait()
            pltpu.sync_copy(rows_v, out_hbm.at[pl.ds(off, chunk)])

    return gather


_gather = None


def _run_gather(features, idx_flat):
    global _gather
    if _gather is None:
        _gather = _make_gather()
    return _gather(features, idx_flat)


def _mm_stats_body(x_ref, w_ref, y_ref, st_ref):
    i = pl.program_id(0)
    y = jnp.dot(x_ref[...], w_ref[...], preferred_element_type=jnp.float32)
    y_ref[...] = y

    @pl.when(i == 0)
    def _():
        st_ref[...] = jnp.zeros((8, _C), jnp.float32)

    st_ref[0:1, :] = st_ref[0:1, :] + jnp.sum(y, axis=0, keepdims=True)
    st_ref[1:2, :] = st_ref[1:2, :] + jnp.sum(y * y, axis=0, keepdims=True)


def _run_mm_stats(x, wt):
    return pl.pallas_call(
        _mm_stats_body,
        grid=(_S * _K // _RB,),
        in_specs=[
            pl.BlockSpec((_RB, _C), lambda i: (i, 0)),
            pl.BlockSpec((_C, _C), lambda i: (0, 0)),
        ],
        out_specs=[
            pl.BlockSpec((_RB, _C), lambda i: (i, 0)),
            pl.BlockSpec((8, _C), lambda i: (0, 0)),
        ],
        out_shape=[
            jax.ShapeDtypeStruct((_S * _K, _C), jnp.float32),
            jax.ShapeDtypeStruct((8, _C), jnp.float32),
        ],
    )(x, wt)


def _bn(y, st_ref, g_ref, b_ref):
    inv_n = jnp.float32(1.0 / (_S * _K))
    mu = st_ref[0:1, :] * inv_n
    var = st_ref[1:2, :] * inv_n - mu * mu
    inv = 1.0 / jnp.sqrt(var + _EPS)
    return jnp.maximum((y - mu) * inv * g_ref[...] + b_ref[...], 0.0)


def _bn_mm_stats_body(y_ref, st_in_ref, g_ref, b_ref, w_ref, y2_ref, st_ref):
    i = pl.program_id(0)
    h = _bn(y_ref[...], st_in_ref, g_ref, b_ref)
    y2 = jnp.dot(h, w_ref[...], preferred_element_type=jnp.float32)
    y2_ref[...] = y2

    @pl.when(i == 0)
    def _():
        st_ref[...] = jnp.zeros((8, _C), jnp.float32)

    st_ref[0:1, :] = st_ref[0:1, :] + jnp.sum(y2, axis=0, keepdims=True)
    st_ref[1:2, :] = st_ref[1:2, :] + jnp.sum(y2 * y2, axis=0, keepdims=True)


def _run_bn_mm_stats(y1, st1, g1, b1, wt2):
    return pl.pallas_call(
        _bn_mm_stats_body,
        grid=(_S * _K // _RB,),
        in_specs=[
            pl.BlockSpec((_RB, _C), lambda i: (i, 0)),
            pl.BlockSpec((8, _C), lambda i: (0, 0)),
            pl.BlockSpec((1, _C), lambda i: (0, 0)),
            pl.BlockSpec((1, _C), lambda i: (0, 0)),
            pl.BlockSpec((_C, _C), lambda i: (0, 0)),
        ],
        out_specs=[
            pl.BlockSpec((_RB, _C), lambda i: (i, 0)),
            pl.BlockSpec((8, _C), lambda i: (0, 0)),
        ],
        out_shape=[
            jax.ShapeDtypeStruct((_S * _K, _C), jnp.float32),
            jax.ShapeDtypeStruct((8, _C), jnp.float32),
        ],
    )(y1, st1, g1, b1, wt2)


def _bn_maxpool_body(y_ref, st_ref, g_ref, b_ref, out_ref):
    j = pl.program_id(0)
    v = _bn(y_ref[...], st_ref, g_ref, b_ref)

    @pl.when(j == 0)
    def _():
        out_ref[...] = v

    @pl.when(j > 0)
    def _():
        out_ref[...] = jnp.maximum(out_ref[...], v)


def _run_bn_maxpool(y2, st2, g2, b2):
    return pl.pallas_call(
        _bn_maxpool_body,
        grid=(_K,),
        in_specs=[
            pl.BlockSpec((_S, _C), lambda j: (j, 0)),
            pl.BlockSpec((8, _C), lambda j: (0, 0)),
            pl.BlockSpec((1, _C), lambda j: (0, 0)),
            pl.BlockSpec((1, _C), lambda j: (0, 0)),
        ],
        out_specs=pl.BlockSpec((_S, _C), lambda j: (0, 0)),
        out_shape=jax.ShapeDtypeStruct((_S, _C), jnp.float32),
    )(y2, st2, g2, b2)


def kernel(points, features, W1, g1, b1, W2, g2, b2):
    ptsT = points.T
    pts3 = ptsT.reshape(3, _G, _G)
    idx_tile, coords = _run_fps(pts3)
    sampled_points = coords.reshape(3, _S).T

    psq = jnp.sum(points ** 2, axis=1)
    psq3 = psq.reshape(_G, 1, _G)
    pad = jnp.zeros((5, _N), jnp.float32)
    ptsb3 = (jnp.concatenate([ptsT, pad], axis=0).astype(jnp.bfloat16)
             .reshape(8, _G, _G).transpose(1, 0, 2))
    sq = jnp.sum(sampled_points ** 2, axis=1, keepdims=True)
    qb = jnp.concatenate(
        [sampled_points, jnp.zeros((_S, 5), jnp.float32)],
        axis=1).astype(jnp.bfloat16)
    knn = _run_topk(sq, qb, ptsb3, psq3)

    idx_flat = knn.T.reshape(-1)
    gathered = _run_gather(features, idx_flat)

    y1, st1 = _run_mm_stats(gathered, W1.T)
    y2, st2 = _run_bn_mm_stats(y1, st1, g1[None, :], b1[None, :], W2.T)
    out_features = _run_bn_maxpool(y2, st2, g2[None, :], b2[None, :])
    return (sampled_points, out_features)


# speedup vs baseline: 1.0980x; 1.0980x over previous
import functools

import jax
import jax.numpy as jnp
from jax import lax
from jax.experimental import pallas as pl
from jax.experimental.pallas import tpu as pltpu
from jax.experimental.pallas import tpu_sc as plsc

_N = 16384
_S = 1024
_K = 16
_C = 256
_EPS = 1e-5
_G = 128
_QB = 128
_RB = 512
_INF = float("inf")
_BIGI = 2**30


def _fps_body(pts_ref, idx_ref, coord_ref):
    px = pts_ref[0]
    py = pts_ref[1]
    pz = pts_ref[2]
    flat = (lax.broadcasted_iota(jnp.int32, (_G, _G), 0) * _G
            + lax.broadcasted_iota(jnp.int32, (_G, _G), 1))
    sub8 = lax.broadcasted_iota(jnp.int32, (8, _G), 0)
    lane8 = lax.broadcasted_iota(jnp.int32, (8, _G), 1)

    def red2(x, red):
        return red(red(x, axis=0, keepdims=True), axis=1, keepdims=True)

    lx = px[0:1, 0:1]
    ly = py[0:1, 0:1]
    lz = pz[0:1, 0:1]
    idx_ref[...] = jnp.zeros((8, _G), jnp.int32)
    c0 = jnp.where(sub8 + lane8 == 0, lx, 0.0)
    c1 = jnp.where(sub8 + lane8 == 0, ly, 0.0)
    c2 = jnp.where(sub8 + lane8 == 0, lz, 0.0)

    flat16 = flat << 16
    def halves(p):
        b = lax.bitcast_convert_type(p, jnp.int32)
        return (flat16 | lax.shift_right_logical(b, 16),
                flat16 | (b & 0xFFFF))
    pxh, pxl = halves(px)
    pyh, pyl = halves(py)
    pzh, pzl = halves(pz)

    def unpack(khi, klo):
        bits = ((khi & 0xFFFF) << 16) | (klo & 0xFFFF)
        return lax.bitcast_convert_type(bits, jnp.float32)

    def body(i, carry):
        dists, lx, ly, lz = carry
        dx = px - lx
        dy = py - ly
        dz = pz - lz
        d = dx * dx + dy * dy + dz * dz
        dists = jnp.minimum(dists, d)
        m = red2(dists, jnp.max)
        sel = dists == m
        kxh = red2(jnp.where(sel, pxh, _BIGI), jnp.min)
        kxl = red2(jnp.where(sel, pxl, _BIGI), jnp.min)
        kyh = red2(jnp.where(sel, pyh, _BIGI), jnp.min)
        kyl = red2(jnp.where(sel, pyl, _BIGI), jnp.min)
        kzh = red2(jnp.where(sel, pzh, _BIGI), jnp.min)
        kzl = red2(jnp.where(sel, pzl, _BIGI), jnp.min)
        idxv = lax.shift_right_logical(kxh, 16)
        nx = unpack(kxh, kxl)
        ny = unpack(kyh, kyl)
        nz = unpack(kzh, kzl)
        sel2 = (sub8 == i // _G) & (lane8 == i % _G)
        idx_ref[...] = jnp.where(sel2, idxv, idx_ref[...])
        coord_ref[0] = jnp.where(sel2, nx, coord_ref[0])
        coord_ref[1] = jnp.where(sel2, ny, coord_ref[1])
        coord_ref[2] = jnp.where(sel2, nz, coord_ref[2])
        return dists, nx, ny, nz

    coord_ref[0] = c0
    coord_ref[1] = c1
    coord_ref[2] = c2
    dists0 = jnp.full((_G, _G), _INF)
    lax.fori_loop(1, _S, body, (dists0, lx, ly, lz))


def _run_fps(pts3):
    return pl.pallas_call(
        _fps_body,
        out_shape=[
            jax.ShapeDtypeStruct((8, _G), jnp.int32),
            jax.ShapeDtypeStruct((3, 8, _G), jnp.float32),
        ],
    )(pts3)


def _topk_body(sq_ref, qb_ref, pts_ref, psq_ref, out_ref, s_ref):
    sq = sq_ref[:, 0:1]
    qb = qb_ref[...]

    def cbody(c, _):
        m = jnp.dot(qb, pts_ref[c], preferred_element_type=jnp.float32)
        s_ref[c] = (sq - 2.0 * m) + psq_ref[c]
        return 0

    lax.fori_loop(0, _N // _G, cbody, 0)

    _U, _A = 8, 4
    lane = lax.broadcasted_iota(jnp.int32, (_QB, _G), 1)
    lv = jnp.full((_QB, 1), -_INF)
    for j in range(_K):
        def fbody(t, st, lv=lv):
            bvs, bcs = list(st[0]), list(st[1])
            for u in range(_U):
                c = t * _U + u
                v = s_ref[c]
                vv = jnp.where(v > lv, v, _INF)
                a = u % _A
                take = vv < bvs[a]
                bvs[a] = jnp.minimum(bvs[a], vv)
                bcs[a] = jnp.where(take, c, bcs[a])
            return tuple(bvs), tuple(bcs)

        bvs, bcs = lax.fori_loop(
            0, _N // _G // _U, fbody,
            (tuple(jnp.full((_QB, _G), _INF) for _ in range(_A)),
             tuple(jnp.zeros((_QB, _G), jnp.int32) for _ in range(_A))))
        bv, bc = bvs[0], bcs[0]
        for a in range(1, _A):
            take = (bvs[a] < bv) | ((bvs[a] == bv) & (bcs[a] < bc))
            bv = jnp.minimum(bv, bvs[a])
            bc = jnp.where(take, bcs[a], bc)
        rv = jnp.min(bv, axis=1, keepdims=True)
        gi = bc * _G + lane
        ri = jnp.min(jnp.where(bv == rv, gi, _BIGI), axis=1, keepdims=True)
        out_ref[:, j:j + 1] = ri
        lv = rv


def _run_topk(sq, qb, ptsb3, psq3):
    return pl.pallas_call(
        _topk_body,
        grid=(_S // _QB,),
        in_specs=[
            pl.BlockSpec((_QB, 1), lambda i: (i, 0)),
            pl.BlockSpec((_QB, 8), lambda i: (i, 0)),
            pl.BlockSpec((_G, 8, _G), lambda i: (0, 0, 0)),
            pl.BlockSpec((_G, 1, _G), lambda i: (0, 0, 0)),
        ],
        out_specs=pl.BlockSpec((_QB, _K), lambda i: (i, 0)),
        out_shape=jax.ShapeDtypeStruct((_S, _K), jnp.int32),
        scratch_shapes=[pltpu.VMEM((_G, _QB, _G), jnp.float32)],
    )(sq, qb, ptsb3, psq3)


def _make_gather():
    info = plsc.get_sparse_core_info()
    nw = info.num_cores * info.num_subcores
    b_per_w = (_S * _K) // nw
    chunk = 128
    mesh = plsc.VectorSubcoreMesh(core_axis_name="c", subcore_axis_name="s")

    @functools.partial(
        pl.kernel, mesh=mesh,
        out_type=jax.ShapeDtypeStruct((_S * _K, _C), jnp.float32),
        scratch_types=[
            pltpu.VMEM((chunk,), jnp.int32),
            pltpu.VMEM((chunk, _C), jnp.float32),
            pltpu.SemaphoreType.DMA,
        ],
    )
    def gather(table_hbm, idx_hbm, out_hbm, idx_v, rows_v, sem):
        wid = lax.axis_index("s") * info.num_cores + lax.axis_index("c")
        base = wid * b_per_w
        for h in range(b_per_w // chunk):
            off = base + h * chunk
            pltpu.sync_copy(idx_hbm.at[pl.ds(off, chunk)], idx_v)
            pltpu.async_copy(table_hbm.at[idx_v], rows_v, sem).wait()
            pltpu.sync_copy(rows_v, out_hbm.at[pl.ds(off, chunk)])

    return gather


_gather = None


def _run_gather(features, idx_flat):
    global _gather
    if _gather is None:
        _gather = _make_gather()
    return _gather(features, idx_flat)


def _mm_stats_body(x_ref, w_ref, y_ref, st_ref):
    i = pl.program_id(0)
    y = jnp.dot(x_ref[...], w_ref[...], preferred_element_type=jnp.float32)
    y_ref[...] = y

    @pl.when(i == 0)
    def _():
        st_ref[...] = jnp.zeros((8, _C), jnp.float32)

    st_ref[0:1, :] = st_ref[0:1, :] + jnp.sum(y, axis=0, keepdims=True)
    st_ref[1:2, :] = st_ref[1:2, :] + jnp.sum(y * y, axis=0, keepdims=True)


def _run_mm_stats(x, wt):
    return pl.pallas_call(
        _mm_stats_body,
        grid=(_S * _K // _RB,),
        in_specs=[
            pl.BlockSpec((_RB, _C), lambda i: (i, 0)),
            pl.BlockSpec((_C, _C), lambda i: (0, 0)),
        ],
        out_specs=[
            pl.BlockSpec((_RB, _C), lambda i: (i, 0)),
            pl.BlockSpec((8, _C), lambda i: (0, 0)),
        ],
        out_shape=[
            jax.ShapeDtypeStruct((_S * _K, _C), jnp.float32),
            jax.ShapeDtypeStruct((8, _C), jnp.float32),
        ],
    )(x, wt)


def _bn(y, st_ref, g_ref, b_ref):
    inv_n = jnp.float32(1.0 / (_S * _K))
    mu = st_ref[0:1, :] * inv_n
    var = st_ref[1:2, :] * inv_n - mu * mu
    inv = 1.0 / jnp.sqrt(var + _EPS)
    return jnp.maximum((y - mu) * inv * g_ref[...] + b_ref[...], 0.0)


def _bn_mm_stats_body(y_ref, st_in_ref, g_ref, b_ref, w_ref, y2_ref, st_ref):
    i = pl.program_id(0)
    h = _bn(y_ref[...], st_in_ref, g_ref, b_ref)
    y2 = jnp.dot(h, w_ref[...], preferred_element_type=jnp.float32)
    y2_ref[...] = y2

    @pl.when(i == 0)
    def _():
        st_ref[...] = jnp.zeros((8, _C), jnp.float32)

    st_ref[0:1, :] = st_ref[0:1, :] + jnp.sum(y2, axis=0, keepdims=True)
    st_ref[1:2, :] = st_ref[1:2, :] + jnp.sum(y2 * y2, axis=0, keepdims=True)


def _run_bn_mm_stats(y1, st1, g1, b1, wt2):
    return pl.pallas_call(
        _bn_mm_stats_body,
        grid=(_S * _K // _RB,),
        in_specs=[
            pl.BlockSpec((_RB, _C), lambda i: (i, 0)),
            pl.BlockSpec((8, _C), lambda i: (0, 0)),
            pl.BlockSpec((1, _C), lambda i: (0, 0)),
            pl.BlockSpec((1, _C), lambda i: (0, 0)),
            pl.BlockSpec((_C, _C), lambda i: (0, 0)),
        ],
        out_specs=[
            pl.BlockSpec((_RB, _C), lambda i: (i, 0)),
            pl.BlockSpec((8, _C), lambda i: (0, 0)),
        ],
        out_shape=[
            jax.ShapeDtypeStruct((_S * _K, _C), jnp.float32),
            jax.ShapeDtypeStruct((8, _C), jnp.float32),
        ],
    )(y1, st1, g1, b1, wt2)


def _bn_maxpool_body(y_ref, st_ref, g_ref, b_ref, out_ref):
    j = pl.program_id(0)
    v = _bn(y_ref[...], st_ref, g_ref, b_ref)

    @pl.when(j == 0)
    def _():
        out_ref[...] = v

    @pl.when(j > 0)
    def _():
        out_ref[...] = jnp.maximum(out_ref[...], v)


def _run_bn_maxpool(y2, st2, g2, b2):
    return pl.pallas_call(
        _bn_maxpool_body,
        grid=(_K,),
        in_specs=[
            pl.BlockSpec((_S, _C), lambda j: (j, 0)),
            pl.BlockSpec((8, _C), lambda j: (0, 0)),
            pl.BlockSpec((1, _C), lambda j: (0, 0)),
            pl.BlockSpec((1, _C), lambda j: (0, 0)),
        ],
        out_specs=pl.BlockSpec((_S, _C), lambda j: (0, 0)),
        out_shape=jax.ShapeDtypeStruct((_S, _C), jnp.float32),
    )(y2, st2, g2, b2)


def kernel(points, features, W1, g1, b1, W2, g2, b2):
    ptsT = points.T
    pts3 = ptsT.reshape(3, _G, _G)
    idx_tile, coords = _run_fps(pts3)
    sampled_points = coords.reshape(3, _S).T

    psq = jnp.sum(points ** 2, axis=1)
    psq3 = psq.reshape(_G, 1, _G)
    pad = jnp.zeros((5, _N), jnp.float32)
    ptsb3 = (jnp.concatenate([ptsT, pad], axis=0).astype(jnp.bfloat16)
             .reshape(8, _G, _G).transpose(1, 0, 2))
    sq = jnp.sum(sampled_points ** 2, axis=1, keepdims=True)
    qb = jnp.concatenate(
        [sampled_points, jnp.zeros((_S, 5), jnp.float32)],
        axis=1).astype(jnp.bfloat16)
    knn = _run_topk(sq, qb, ptsb3, psq3)

    idx_flat = knn.T.reshape(-1)
    gathered = _run_gather(features, idx_flat)

    y1, st1 = _run_mm_stats(gathered, W1.T)
    y2, st2 = _run_bn_mm_stats(y1, st1, g1[None, :], b1[None, :], W2.T)
    out_features = _run_bn_maxpool(y2, st2, g2[None, :], b2[None, :])
    return (sampled_points, out_features)
